# baseline (speedup 1.0000x reference)
import functools

import jax
import jax.numpy as jnp
from jax import lax
from jax.experimental import pallas as pl
from jax.experimental.pallas import tpu as pltpu
from jax.experimental.pallas import tpu_sc as plsc

N = 10000
E = 320000
D = 128
B = 64
DC = 16

NC = 2
NS = 16
NT = NC * NS
EPT = E // NT
CH = 80
NCHUNK = EPT // CH
RPT = N // NS
ZR = 25

RB = 1000
G1 = N // RB

_HIGH = lax.Precision.HIGHEST


@functools.lru_cache(maxsize=None)
def _make_sc_agg(width):
    mesh = plsc.VectorSubcoreMesh(core_axis_name="c", subcore_axis_name="s")

    NR = 3
    NI = 6

    @functools.partial(
        pl.kernel,
        out_type=jax.ShapeDtypeStruct((NC, N, width), jnp.float32),
        mesh=mesh,
        compiler_params=pltpu.CompilerParams(use_tc_tiling_on_sc=False),
        scratch_types=[
            pltpu.VMEM((NI, 2, CH), jnp.int32),
            pltpu.VMEM((NR, CH, width), jnp.float32),
            pltpu.VMEM((ZR, width), jnp.float32),
            pltpu.VMEM_SHARED((N, width), jnp.float32),
        ] + [pltpu.SemaphoreType.DMA] * (NI + 2 * NR),
    )
    def agg(xa_hbm, ei_hbm, out_hbm, idxr, rowsr, zbuf, acc, *sems):
        c = lax.axis_index("c")
        s = lax.axis_index("s")
        tid = c * NS + s
        isems = sems[:NI]
        gsems = sems[NI:NI + NR]
        ssems = sems[NI + NR:]

        def idx_start(ch, ia):
            pltpu.async_copy(ei_hbm.at[tid, ch], idxr.at[ia], isems[ia])

        def idx_wait(ch, ia):
            pltpu.make_async_copy(ei_hbm.at[tid, ch], idxr.at[ia],
                                  isems[ia]).wait()

        def g_start(rs, ia):
            pltpu.async_copy(xa_hbm.at[idxr.at[ia, 0]], rowsr.at[rs],
                             gsems[rs])

        def g_wait(rs, ia):
            pltpu.make_async_copy(xa_hbm.at[idxr.at[ia, 0]], rowsr.at[rs],
                                  gsems[rs]).wait()

        def s_start(rs, ia):
            pltpu.async_copy(rowsr.at[rs], acc.at[idxr.at[ia, 1]],
                             ssems[rs], add=True)

        def s_wait(rs, ia):
            pltpu.make_async_copy(rowsr.at[rs], acc.at[idxr.at[ia, 1]],
                                  ssems[rs]).wait()

        def step(k, j, guard):
            a, ia = j % NR, j % NI
            b, ib = (j + 1) % NR, (j + 1) % NI
            g_wait(a, ia)
            s_start(a, ia)
            if not guard or k + 1 < NCHUNK:
                if not guard or k - 2 >= 0:
                    s_wait(b, (j + 4) % NI)
                idx_wait(k + 1, ib)
                g_start(b, ib)
            if not guard or k + 4 < NCHUNK:
                idx_start(k + 4, (j + 4) % NI)

        for k in range(4):
            idx_start(k, k)
        idx_wait(0, 0)
        g_start(0, 0)

        nv = width // 16

        def zrow(i, carry):
            for j in range(nv):
                zbuf[i, pl.ds(j * 16, 16)] = jnp.zeros((16,), jnp.float32)
            return carry

        lax.fori_loop(0, ZR, zrow, 0)
        for r in range(RPT // ZR):
            pltpu.sync_copy(zbuf, acc.at[pl.ds(s * RPT + r * ZR, ZR)])
        plsc.subcore_barrier()

        for k in range(6):
            step(k, k, True)

        def body(i, carry):
            k0 = 6 * i
            for j in range(6):
                step(k0 + j, j, False)
            return carry

        nbody = (NCHUNK - 10) // 6
        lax.fori_loop(1, nbody + 1, body, 0)

        for k in range(6 * (nbody + 1), NCHUNK):
            step(k, k % 6, True)

        for k in range(NCHUNK - NR, NCHUNK):
            s_wait(k % NR, k % NI)
        plsc.subcore_barrier()

        pltpu.sync_copy(acc.at[pl.ds(s * RPT, RPT)],
                        out_hbm.at[c, pl.ds(s * RPT, RPT)])

    return agg


def _tc_layer1(parts, x, WlT, WrT, b1):
    def body(p_ref, x_ref, wl_ref, wr_ref, b_ref, h_ref, rinv_ref):
        agg = p_ref[0, :, :D] + p_ref[1, :, :D]
        cnt = p_ref[0, :, D:D + 1] + p_ref[1, :, D:D + 1]
        rinv = 1.0 / jnp.maximum(cnt, 1.0)
        mean = agg * rinv
        h = jnp.dot(mean, wl_ref[...], precision=_HIGH,
                    preferred_element_type=jnp.float32)
        h += jnp.dot(x_ref[...], wr_ref[...], precision=_HIGH,
                     preferred_element_type=jnp.float32)
        h += b_ref[...]
        h_ref[...] = jnp.maximum(h, 0.0)
        rinv_ref[...] = jnp.broadcast_to(rinv, (RB, D))

    return pl.pallas_call(
        body,
        grid=(G1,),
        in_specs=[
            pl.BlockSpec((NC, RB, D + DC), lambda i: (0, i, 0)),
            pl.BlockSpec((RB, D), lambda i: (i, 0)),
            pl.BlockSpec((D, D), lambda i: (0, 0)),
            pl.BlockSpec((D, D), lambda i: (0, 0)),
            pl.BlockSpec((1, D), lambda i: (0, 0)),
        ],
        out_specs=[
            pl.BlockSpec((RB, D), lambda i: (i, 0)),
            pl.BlockSpec((RB, D), lambda i: (i, 0)),
        ],
        out_shape=[
            jax.ShapeDtypeStruct((N, D), jnp.float32),
            jax.ShapeDtypeStruct((N, D), jnp.float32),
        ],
    )(parts, x, WlT, WrT, b1)


def _tc_layer2(parts2, rinv, h, seg3, WlT, WrT, b2):
    def body(p_ref, rinv_ref, h_ref, seg_ref, wl_ref, wr_ref, b_ref,
             out_ref, pm, ph, po):
        i = pl.program_id(0)

        @pl.when(i == 0)
        def _():
            pm[...] = jnp.zeros((B, D), jnp.float32)
            ph[...] = jnp.zeros((B, D), jnp.float32)
            po[...] = jnp.zeros((B, D), jnp.float32)

        mean2 = (p_ref[0] + p_ref[1]) * rinv_ref[...]
        seg = seg_ref[0]
        ohT = (lax.broadcasted_iota(jnp.int32, (B, RB), 0) == seg
               ).astype(jnp.float32)
        pm[...] += jnp.dot(ohT, mean2, precision=_HIGH,
                           preferred_element_type=jnp.float32)
        ph[...] += jnp.dot(ohT, h_ref[...], precision=_HIGH,
                           preferred_element_type=jnp.float32)
        po[...] += jnp.dot(ohT, jnp.ones((RB, D), jnp.float32),
                           precision=_HIGH, preferred_element_type=jnp.float32)

        @pl.when(i == G1 - 1)
        def _():
            out_ref[...] = (
                jnp.dot(pm[...], wl_ref[...], precision=_HIGH,
                        preferred_element_type=jnp.float32)
                + jnp.dot(ph[...], wr_ref[...], precision=_HIGH,
                          preferred_element_type=jnp.float32)
                + po[...] * b_ref[...])

    return pl.pallas_call(
        body,
        grid=(G1,),
        in_specs=[
            pl.BlockSpec((NC, RB, D), lambda i: (0, i, 0)),
            pl.BlockSpec((RB, D), lambda i: (i, 0)),
            pl.BlockSpec((RB, D), lambda i: (i, 0)),
            pl.BlockSpec((1, 1, RB), lambda i: (i, 0, 0)),
            pl.BlockSpec((D, D), lambda i: (0, 0)),
            pl.BlockSpec((D, D), lambda i: (0, 0)),
            pl.BlockSpec((1, D), lambda i: (0, 0)),
        ],
        out_specs=pl.BlockSpec((B, D), lambda i: (0, 0)),
        out_shape=jax.ShapeDtypeStruct((B, D), jnp.float32),
        scratch_shapes=[
            pltpu.VMEM((B, D), jnp.float32),
            pltpu.VMEM((B, D), jnp.float32),
            pltpu.VMEM((B, D), jnp.float32),
        ],
    )(parts2, rinv, h, seg3, WlT, WrT, b2)


def kernel(x, edge_index, batch, batch_size, Wl1, Wr1, b1, Wl2, Wr2, b2):
    x = x.astype(jnp.float32)
    ei = jnp.stack(
        [edge_index[0].astype(jnp.int32).reshape(NT, NCHUNK, CH),
         edge_index[1].astype(jnp.int32).reshape(NT, NCHUNK, CH)], axis=2)
    seg3 = jnp.minimum(batch, batch_size - 1).astype(jnp.int32).reshape(G1, 1, RB)

    xa = jnp.concatenate([x, jnp.ones((N, DC), jnp.float32)], axis=1)
    parts1 = _make_sc_agg(D + DC)(xa, ei)
    h, rinv = _tc_layer1(parts1, x, Wl1.T, Wr1.T, b1.reshape(1, D))

    parts2 = _make_sc_agg(D)(h, ei)
    return _tc_layer2(parts2, rinv, h, seg3, Wl2.T, Wr2.T, b2.reshape(1, D))


# speedup vs baseline: 1.2711x; 1.2711x over previous
import functools

import jax
import jax.numpy as jnp
from jax import lax
from jax.experimental import pallas as pl
from jax.experimental.pallas import tpu as pltpu
from jax.experimental.pallas import tpu_sc as plsc

N = 10000
E = 320000
D = 128
B = 64
DC = 16

NC = 2
NS = 16
NT = NC * NS
EPT = E // NT
CH = 40
NCHUNK = EPT // CH
RPT = N // NS
ZR = 25

RB = 1000
G1 = N // RB

_HIGH = lax.Precision.HIGHEST


@functools.lru_cache(maxsize=None)
def _make_sc_agg(width):
    mesh = plsc.VectorSubcoreMesh(core_axis_name="c", subcore_axis_name="s")

    NR = 6
    NI = 12
    LG = 4
    LI = 8

    @functools.partial(
        pl.kernel,
        out_type=jax.ShapeDtypeStruct((NC, N, width), jnp.float32),
        mesh=mesh,
        compiler_params=pltpu.CompilerParams(use_tc_tiling_on_sc=False),
        scratch_types=[
            pltpu.VMEM((NI, 2, CH), jnp.int32),
            pltpu.VMEM((NR, CH, width), jnp.float32),
            pltpu.VMEM((ZR, width), jnp.float32),
            pltpu.VMEM_SHARED((N, width), jnp.float32),
        ] + [pltpu.SemaphoreType.DMA] * (NI + 2 * NR),
    )
    def agg(xa_hbm, ei_hbm, out_hbm, idxr, rowsr, zbuf, acc, *sems):
        c = lax.axis_index("c")
        s = lax.axis_index("s")
        tid = c * NS + s
        isems = sems[:NI]
        gsems = sems[NI:NI + NR]
        ssems = sems[NI + NR:]

        def idx_start(ch, ia):
            pltpu.async_copy(ei_hbm.at[tid, ch], idxr.at[ia], isems[ia])

        def idx_wait(ch, ia):
            pltpu.make_async_copy(ei_hbm.at[tid, ch], idxr.at[ia],
                                  isems[ia]).wait()

        def g_start(rs, ia):
            pltpu.async_copy(xa_hbm.at[idxr.at[ia, 0]], rowsr.at[rs],
                             gsems[rs])

        def g_wait(rs, ia):
            pltpu.make_async_copy(xa_hbm.at[idxr.at[ia, 0]], rowsr.at[rs],
                                  gsems[rs]).wait()

        def s_start(rs, ia):
            pltpu.async_copy(rowsr.at[rs], acc.at[idxr.at[ia, 1]],
                             ssems[rs], add=True)

        def s_wait(rs, ia):
            pltpu.make_async_copy(rowsr.at[rs], acc.at[idxr.at[ia, 1]],
                                  ssems[rs]).wait()

        def step(k, j, guard):
            g_wait(j % NR, j % NI)
            s_start(j % NR, j % NI)
            if not guard or k + LG < NCHUNK:
                if not guard or k - (NR - LG) >= 0:
                    s_wait((j + LG) % NR, (j + LG - NR) % NI)
                idx_wait(k + LG, (j + LG) % NI)
                g_start((j + LG) % NR, (j + LG) % NI)
            if not guard or k + LI < NCHUNK:
                idx_start(k + LI, (j + LI) % NI)

        for k in range(LI):
            idx_start(k, k)
        for k in range(LG):
            idx_wait(k, k)
            g_start(k % NR, k)

        nv = width // 16

        def zrow(i, carry):
            for j in range(nv):
                zbuf[i, pl.ds(j * 16, 16)] = jnp.zeros((16,), jnp.float32)
            return carry

        lax.fori_loop(0, ZR, zrow, 0)
        for r in range(RPT // ZR):
            pltpu.sync_copy(zbuf, acc.at[pl.ds(s * RPT + r * ZR, ZR)])
        plsc.subcore_barrier()

        for k in range(NI):
            step(k, k, True)

        def body(i, carry):
            k0 = NI * i
            for j in range(NI):
                step(k0 + j, j, False)
            return carry

        nbody = (NCHUNK - LI - NI) // NI
        lax.fori_loop(1, nbody + 1, body, 0)

        for k in range(NI * (nbody + 1), NCHUNK):
            step(k, k % NI, True)

        for k in range(NCHUNK - NR, NCHUNK):
            s_wait(k % NR, k % NI)
        plsc.subcore_barrier()

        pltpu.sync_copy(acc.at[pl.ds(s * RPT, RPT)],
                        out_hbm.at[c, pl.ds(s * RPT, RPT)])

    return agg


def _tc_layer1(parts, x, WlT, WrT, b1):
    def body(p_ref, x_ref, wl_ref, wr_ref, b_ref, h_ref, rinv_ref):
        agg = p_ref[0, :, :D] + p_ref[1, :, :D]
        cnt = p_ref[0, :, D:D + 1] + p_ref[1, :, D:D + 1]
        rinv = 1.0 / jnp.maximum(cnt, 1.0)
        mean = agg * rinv
        h = jnp.dot(mean, wl_ref[...], precision=_HIGH,
                    preferred_element_type=jnp.float32)
        h += jnp.dot(x_ref[...], wr_ref[...], precision=_HIGH,
                     preferred_element_type=jnp.float32)
        h += b_ref[...]
        h_ref[...] = jnp.maximum(h, 0.0)
        rinv_ref[...] = jnp.broadcast_to(rinv, (RB, D))

    return pl.pallas_call(
        body,
        grid=(G1,),
        in_specs=[
            pl.BlockSpec((NC, RB, D + DC), lambda i: (0, i, 0)),
            pl.BlockSpec((RB, D), lambda i: (i, 0)),
            pl.BlockSpec((D, D), lambda i: (0, 0)),
            pl.BlockSpec((D, D), lambda i: (0, 0)),
            pl.BlockSpec((1, D), lambda i: (0, 0)),
        ],
        out_specs=[
            pl.BlockSpec((RB, D), lambda i: (i, 0)),
            pl.BlockSpec((RB, D), lambda i: (i, 0)),
        ],
        out_shape=[
            jax.ShapeDtypeStruct((N, D), jnp.float32),
            jax.ShapeDtypeStruct((N, D), jnp.float32),
        ],
    )(parts, x, WlT, WrT, b1)


def _tc_layer2(parts2, rinv, h, seg3, WlT, WrT, b2):
    def body(p_ref, rinv_ref, h_ref, seg_ref, wl_ref, wr_ref, b_ref,
             out_ref, pm, ph, po):
        i = pl.program_id(0)

        @pl.when(i == 0)
        def _():
            pm[...] = jnp.zeros((B, D), jnp.float32)
            ph[...] = jnp.zeros((B, D), jnp.float32)
            po[...] = jnp.zeros((B, D), jnp.float32)

        mean2 = (p_ref[0] + p_ref[1]) * rinv_ref[...]
        seg = seg_ref[0]
        ohT = (lax.broadcasted_iota(jnp.int32, (B, RB), 0) == seg
               ).astype(jnp.float32)
        pm[...] += jnp.dot(ohT, mean2, precision=_HIGH,
                           preferred_element_type=jnp.float32)
        ph[...] += jnp.dot(ohT, h_ref[...], precision=_HIGH,
                           preferred_element_type=jnp.float32)
        po[...] += jnp.dot(ohT, jnp.ones((RB, D), jnp.float32),
                           precision=_HIGH, preferred_element_type=jnp.float32)

        @pl.when(i == G1 - 1)
        def _():
            out_ref[...] = (
                jnp.dot(pm[...], wl_ref[...], precision=_HIGH,
                        preferred_element_type=jnp.float32)
                + jnp.dot(ph[...], wr_ref[...], precision=_HIGH,
                          preferred_element_type=jnp.float32)
                + po[...] * b_ref[...])

    return pl.pallas_call(
        body,
        grid=(G1,),
        in_specs=[
            pl.BlockSpec((NC, RB, D), lambda i: (0, i, 0)),
            pl.BlockSpec((RB, D), lambda i: (i, 0)),
            pl.BlockSpec((RB, D), lambda i: (i, 0)),
            pl.BlockSpec((1, 1, RB), lambda i: (i, 0, 0)),
            pl.BlockSpec((D, D), lambda i: (0, 0)),
            pl.BlockSpec((D, D), lambda i: (0, 0)),
            pl.BlockSpec((1, D), lambda i: (0, 0)),
        ],
        out_specs=pl.BlockSpec((B, D), lambda i: (0, 0)),
        out_shape=jax.ShapeDtypeStruct((B, D), jnp.float32),
        scratch_shapes=[
            pltpu.VMEM((B, D), jnp.float32),
            pltpu.VMEM((B, D), jnp.float32),
            pltpu.VMEM((B, D), jnp.float32),
        ],
    )(parts2, rinv, h, seg3, WlT, WrT, b2)


def kernel(x, edge_index, batch, batch_size, Wl1, Wr1, b1, Wl2, Wr2, b2):
    x = x.astype(jnp.float32)
    ei = jnp.stack(
        [edge_index[0].astype(jnp.int32).reshape(NT, NCHUNK, CH),
         edge_index[1].astype(jnp.int32).reshape(NT, NCHUNK, CH)], axis=2)
    seg3 = jnp.minimum(batch, batch_size - 1).astype(jnp.int32).reshape(G1, 1, RB)

    xa = jnp.concatenate([x, jnp.ones((N, DC), jnp.float32)], axis=1)
    parts1 = _make_sc_agg(D + DC)(xa, ei)
    h, rinv = _tc_layer1(parts1, x, Wl1.T, Wr1.T, b1.reshape(1, D))

    parts2 = _make_sc_agg(D)(h, ei)
    return _tc_layer2(parts2, rinv, h, seg3, Wl2.T, Wr2.T, b2.reshape(1, D))


# speedup vs baseline: 1.2944x; 1.0184x over previous
import functools

import jax
import jax.numpy as jnp
from jax import lax
from jax.experimental import pallas as pl
from jax.experimental.pallas import tpu as pltpu
from jax.experimental.pallas import tpu_sc as plsc

N = 10000
E = 320000
D = 128
B = 64
DC = 16

NC = 2
NS = 16
NT = NC * NS
EPT = E // NT
CH = 40
NCHUNK = EPT // CH
RPT = N // NS
ZR = 25

RB = 1000
G1 = N // RB

_HIGH = lax.Precision.HIGHEST


@functools.lru_cache(maxsize=None)
def _make_sc_agg(width):
    mesh = plsc.VectorSubcoreMesh(core_axis_name="c", subcore_axis_name="s")

    NR = 6
    NI = 12
    LG = 5
    LI = 8

    @functools.partial(
        pl.kernel,
        out_type=jax.ShapeDtypeStruct((NC, N, width), jnp.float32),
        mesh=mesh,
        compiler_params=pltpu.CompilerParams(use_tc_tiling_on_sc=False),
        scratch_types=[
            pltpu.VMEM((NI, 2, CH), jnp.int32),
            pltpu.VMEM((NR, CH, width), jnp.float32),
            pltpu.VMEM((ZR, width), jnp.float32),
            pltpu.VMEM_SHARED((N, width), jnp.float32),
        ] + [pltpu.SemaphoreType.DMA] * (NI + 2 * NR),
    )
    def agg(xa_hbm, ei_hbm, out_hbm, idxr, rowsr, zbuf, acc, *sems):
        c = lax.axis_index("c")
        s = lax.axis_index("s")
        tid = c * NS + s
        isems = sems[:NI]
        gsems = sems[NI:NI + NR]
        ssems = sems[NI + NR:]

        def idx_start(ch, ia):
            pltpu.async_copy(ei_hbm.at[tid, ch], idxr.at[ia], isems[ia])

        def idx_wait(ch, ia):
            pltpu.make_async_copy(ei_hbm.at[tid, ch], idxr.at[ia],
                                  isems[ia]).wait()

        def g_start(rs, ia):
            pltpu.async_copy(xa_hbm.at[idxr.at[ia, 0]], rowsr.at[rs],
                             gsems[rs])

        def g_wait(rs, ia):
            pltpu.make_async_copy(xa_hbm.at[idxr.at[ia, 0]], rowsr.at[rs],
                                  gsems[rs]).wait()

        def s_start(rs, ia):
            pltpu.async_copy(rowsr.at[rs], acc.at[idxr.at[ia, 1]],
                             ssems[rs], add=True)

        def s_wait(rs, ia):
            pltpu.make_async_copy(rowsr.at[rs], acc.at[idxr.at[ia, 1]],
                                  ssems[rs]).wait()

        def step(k, j, guard):
            g_wait(j % NR, j % NI)
            s_start(j % NR, j % NI)
            if not guard or k + LG < NCHUNK:
                if not guard or k - (NR - LG) >= 0:
                    s_wait((j + LG) % NR, (j + LG - NR) % NI)
                idx_wait(k + LG, (j + LG) % NI)
                g_start((j + LG) % NR, (j + LG) % NI)
            if not guard or k + LI < NCHUNK:
                idx_start(k + LI, (j + LI) % NI)

        for k in range(LI):
            idx_start(k, k)
        for k in range(LG):
            idx_wait(k, k)
            g_start(k % NR, k)

        nv = width // 16

        def zrow(i, carry):
            for j in range(nv):
                zbuf[i, pl.ds(j * 16, 16)] = jnp.zeros((16,), jnp.float32)
            return carry

        lax.fori_loop(0, ZR, zrow, 0)
        for r in range(RPT // ZR):
            pltpu.sync_copy(zbuf, acc.at[pl.ds(s * RPT + r * ZR, ZR)])
        plsc.subcore_barrier()

        for k in range(NI):
            step(k, k, True)

        def body(i, carry):
            k0 = NI * i
            for j in range(NI):
                step(k0 + j, j, False)
            return carry

        nbody = (NCHUNK - LI - NI) // NI
        lax.fori_loop(1, nbody + 1, body, 0)

        for k in range(NI * (nbody + 1), NCHUNK):
            step(k, k % NI, True)

        for k in range(NCHUNK - NR, NCHUNK):
            s_wait(k % NR, k % NI)
        plsc.subcore_barrier()

        pltpu.sync_copy(acc.at[pl.ds(s * RPT, RPT)],
                        out_hbm.at[c, pl.ds(s * RPT, RPT)])

    return agg


def _tc_layer1(parts, x, WlT, WrT, b1):
    def body(p_ref, x_ref, wl_ref, wr_ref, b_ref, h_ref, rinv_ref):
        agg = p_ref[0, :, :D] + p_ref[1, :, :D]
        cnt = p_ref[0, :, D:D + 1] + p_ref[1, :, D:D + 1]
        rinv = 1.0 / jnp.maximum(cnt, 1.0)
        mean = agg * rinv
        h = jnp.dot(mean, wl_ref[...], precision=_HIGH,
                    preferred_element_type=jnp.float32)
        h += jnp.dot(x_ref[...], wr_ref[...], precision=_HIGH,
                     preferred_element_type=jnp.float32)
        h += b_ref[...]
        h_ref[...] = jnp.maximum(h, 0.0)
        rinv_ref[...] = jnp.broadcast_to(rinv, (RB, D))

    return pl.pallas_call(
        body,
        grid=(G1,),
        in_specs=[
            pl.BlockSpec((NC, RB, D + DC), lambda i: (0, i, 0)),
            pl.BlockSpec((RB, D), lambda i: (i, 0)),
            pl.BlockSpec((D, D), lambda i: (0, 0)),
            pl.BlockSpec((D, D), lambda i: (0, 0)),
            pl.BlockSpec((1, D), lambda i: (0, 0)),
        ],
        out_specs=[
            pl.BlockSpec((RB, D), lambda i: (i, 0)),
            pl.BlockSpec((RB, D), lambda i: (i, 0)),
        ],
        out_shape=[
            jax.ShapeDtypeStruct((N, D), jnp.float32),
            jax.ShapeDtypeStruct((N, D), jnp.float32),
        ],
    )(parts, x, WlT, WrT, b1)


def _tc_layer2(parts2, rinv, h, seg3, WlT, WrT, b2):
    def body(p_ref, rinv_ref, h_ref, seg_ref, wl_ref, wr_ref, b_ref,
             out_ref, pm, ph, po):
        i = pl.program_id(0)

        @pl.when(i == 0)
        def _():
            pm[...] = jnp.zeros((B, D), jnp.float32)
            ph[...] = jnp.zeros((B, D), jnp.float32)
            po[...] = jnp.zeros((B, D), jnp.float32)

        mean2 = (p_ref[0] + p_ref[1]) * rinv_ref[...]
        seg = seg_ref[0]
        ohT = (lax.broadcasted_iota(jnp.int32, (B, RB), 0) == seg
               ).astype(jnp.float32)
        pm[...] += jnp.dot(ohT, mean2, precision=_HIGH,
                           preferred_element_type=jnp.float32)
        ph[...] += jnp.dot(ohT, h_ref[...], precision=_HIGH,
                           preferred_element_type=jnp.float32)
        po[...] += jnp.dot(ohT, jnp.ones((RB, D), jnp.float32),
                           precision=_HIGH, preferred_element_type=jnp.float32)

        @pl.when(i == G1 - 1)
        def _():
            out_ref[...] = (
                jnp.dot(pm[...], wl_ref[...], precision=_HIGH,
                        preferred_element_type=jnp.float32)
                + jnp.dot(ph[...], wr_ref[...], precision=_HIGH,
                          preferred_element_type=jnp.float32)
                + po[...] * b_ref[...])

    return pl.pallas_call(
        body,
        grid=(G1,),
        in_specs=[
            pl.BlockSpec((NC, RB, D), lambda i: (0, i, 0)),
            pl.BlockSpec((RB, D), lambda i: (i, 0)),
            pl.BlockSpec((RB, D), lambda i: (i, 0)),
            pl.BlockSpec((1, 1, RB), lambda i: (i, 0, 0)),
            pl.BlockSpec((D, D), lambda i: (0, 0)),
            pl.BlockSpec((D, D), lambda i: (0, 0)),
            pl.BlockSpec((1, D), lambda i: (0, 0)),
        ],
        out_specs=pl.BlockSpec((B, D), lambda i: (0, 0)),
        out_shape=jax.ShapeDtypeStruct((B, D), jnp.float32),
        scratch_shapes=[
            pltpu.VMEM((B, D), jnp.float32),
            pltpu.VMEM((B, D), jnp.float32),
            pltpu.VMEM((B, D), jnp.float32),
        ],
    )(parts2, rinv, h, seg3, WlT, WrT, b2)


def kernel(x, edge_index, batch, batch_size, Wl1, Wr1, b1, Wl2, Wr2, b2):
    x = x.astype(jnp.float32)
    ei = jnp.stack(
        [edge_index[0].astype(jnp.int32).reshape(NT, NCHUNK, CH),
         edge_index[1].astype(jnp.int32).reshape(NT, NCHUNK, CH)], axis=2)
    seg3 = jnp.minimum(batch, batch_size - 1).astype(jnp.int32).reshape(G1, 1, RB)

    xa = jnp.concatenate([x, jnp.ones((N, DC), jnp.float32)], axis=1)
    parts1 = _make_sc_agg(D + DC)(xa, ei)
    h, rinv = _tc_layer1(parts1, x, Wl1.T, Wr1.T, b1.reshape(1, D))

    parts2 = _make_sc_agg(D)(h, ei)
    return _tc_layer2(parts2, rinv, h, seg3, Wl2.T, Wr2.T, b2.reshape(1, D))


# speedup vs baseline: 1.3180x; 1.0182x over previous
import functools

import jax
import jax.numpy as jnp
from jax import lax
from jax.experimental import pallas as pl
from jax.experimental.pallas import tpu as pltpu
from jax.experimental.pallas import tpu_sc as plsc

N = 10000
E = 320000
D = 128
B = 64
DC = 16

NC = 2
NS = 16
NT = NC * NS
EPT = E // NT
CH = 40
NCHUNK = EPT // CH
RPT = N // NS
ZR = 25

RB = 1000
G1 = N // RB

_HIGH = lax.Precision.HIGHEST


@functools.lru_cache(maxsize=None)
def _make_sc_agg(width):
    mesh = plsc.VectorSubcoreMesh(core_axis_name="c", subcore_axis_name="s")

    NR = 6
    NI = 12
    LG = 5
    LI = 8

    @functools.partial(
        pl.kernel,
        out_type=jax.ShapeDtypeStruct((NC, N, width), jnp.float32),
        mesh=mesh,
        compiler_params=pltpu.CompilerParams(use_tc_tiling_on_sc=False),
        scratch_types=[
            pltpu.VMEM((NI, 2, CH), jnp.int32),
            pltpu.VMEM((NR, CH, width), jnp.float32),
            pltpu.VMEM((ZR, width), jnp.float32),
            pltpu.VMEM_SHARED((N, width), jnp.float32),
        ] + [pltpu.SemaphoreType.DMA] * (NI + 2 * NR),
    )
    def agg(xa_hbm, ei_hbm, out_hbm, idxr, rowsr, zbuf, acc, *sems):
        c = lax.axis_index("c")
        s = lax.axis_index("s")
        tid = c * NS + s
        isems = sems[:NI]
        gsems = sems[NI:NI + NR]
        ssems = sems[NI + NR:]

        def idx_start(ch, ia):
            pltpu.async_copy(ei_hbm.at[tid, ch], idxr.at[ia], isems[ia])

        def idx_wait(ch, ia):
            pltpu.make_async_copy(ei_hbm.at[tid, ch], idxr.at[ia],
                                  isems[ia]).wait()

        def g_start(rs, ia):
            pltpu.async_copy(xa_hbm.at[idxr.at[ia, 0]], rowsr.at[rs],
                             gsems[rs])

        def g_wait(rs, ia):
            pltpu.make_async_copy(xa_hbm.at[idxr.at[ia, 0]], rowsr.at[rs],
                                  gsems[rs]).wait()

        def s_start(rs, ia):
            pltpu.async_copy(rowsr.at[rs], acc.at[idxr.at[ia, 1]],
                             ssems[rs], add=True)

        def s_wait(rs, ia):
            pltpu.make_async_copy(rowsr.at[rs], acc.at[idxr.at[ia, 1]],
                                  ssems[rs]).wait()

        def step(k, j, guard):
            g_wait(j % NR, j % NI)
            s_start(j % NR, j % NI)
            if not guard or k + LG < NCHUNK:
                if not guard or k - (NR - LG) >= 0:
                    s_wait((j + LG) % NR, (j + LG - NR) % NI)
                idx_wait(k + LG, (j + LG) % NI)
                g_start((j + LG) % NR, (j + LG) % NI)
            if not guard or k + LI < NCHUNK:
                idx_start(k + LI, (j + LI) % NI)

        for k in range(LI):
            idx_start(k, k)
        for k in range(LG):
            idx_wait(k, k)
            g_start(k % NR, k)

        nv = width // 16

        def zrow(i, carry):
            for j in range(nv):
                zbuf[i, pl.ds(j * 16, 16)] = jnp.zeros((16,), jnp.float32)
            return carry

        lax.fori_loop(0, ZR, zrow, 0)
        for r in range(RPT // ZR):
            pltpu.sync_copy(zbuf, acc.at[pl.ds(s * RPT + r * ZR, ZR)])
        plsc.subcore_barrier()

        for k in range(NI):
            step(k, k, True)

        def body(i, carry):
            k0 = NI * i
            for j in range(NI):
                step(k0 + j, j, False)
            return carry

        nbody = (NCHUNK - LI - NI) // NI
        lax.fori_loop(1, nbody + 1, body, 0)

        for k in range(NI * (nbody + 1), NCHUNK):
            step(k, k % NI, True)

        for k in range(NCHUNK - NR, NCHUNK):
            s_wait(k % NR, k % NI)
        plsc.subcore_barrier()

        pltpu.sync_copy(acc.at[pl.ds(s * RPT, RPT)],
                        out_hbm.at[c, pl.ds(s * RPT, RPT)])

    return agg


def _tc_xr(x, WrT, b):
    def body(x_ref, wr_ref, b_ref, xr_ref):
        xr_ref[...] = jnp.dot(x_ref[...], wr_ref[...], precision=_HIGH,
                              preferred_element_type=jnp.float32) + b_ref[...]

    return pl.pallas_call(
        body,
        grid=(G1,),
        in_specs=[
            pl.BlockSpec((RB, D), lambda i: (i, 0)),
            pl.BlockSpec((D, D), lambda i: (0, 0)),
            pl.BlockSpec((1, D), lambda i: (0, 0)),
        ],
        out_specs=pl.BlockSpec((RB, D), lambda i: (i, 0)),
        out_shape=jax.ShapeDtypeStruct((N, D), jnp.float32),
    )(x, WrT, b)


def _tc_layer1(parts, xr, WlT):
    def body(p_ref, xr_ref, wl_ref, h_ref, rinv_ref):
        agg = p_ref[0, :, :D] + p_ref[1, :, :D]
        cnt = p_ref[0, :, D:D + 1] + p_ref[1, :, D:D + 1]
        rinv = 1.0 / jnp.maximum(cnt, 1.0)
        mean = agg * rinv
        h = jnp.dot(mean, wl_ref[...], precision=_HIGH,
                    preferred_element_type=jnp.float32) + xr_ref[...]
        h_ref[...] = jnp.maximum(h, 0.0)
        rinv_ref[...] = jnp.broadcast_to(rinv, (RB, D))

    return pl.pallas_call(
        body,
        grid=(G1,),
        in_specs=[
            pl.BlockSpec((NC, RB, D + DC), lambda i: (0, i, 0)),
            pl.BlockSpec((RB, D), lambda i: (i, 0)),
            pl.BlockSpec((D, D), lambda i: (0, 0)),
        ],
        out_specs=[
            pl.BlockSpec((RB, D), lambda i: (i, 0)),
            pl.BlockSpec((RB, D), lambda i: (i, 0)),
        ],
        out_shape=[
            jax.ShapeDtypeStruct((N, D), jnp.float32),
            jax.ShapeDtypeStruct((N, D), jnp.float32),
        ],
    )(parts, xr, WlT)


def _tc_pool_h(h, seg3, WrT, b2):
    def body(h_ref, seg_ref, wr_ref, b_ref, hc_ref, ph, po):
        i = pl.program_id(0)

        @pl.when(i == 0)
        def _():
            ph[...] = jnp.zeros((B, D), jnp.float32)
            po[...] = jnp.zeros((B, D), jnp.float32)

        seg = seg_ref[0]
        ohT = (lax.broadcasted_iota(jnp.int32, (B, RB), 0) == seg
               ).astype(jnp.float32)
        ph[...] += jnp.dot(ohT, h_ref[...], precision=_HIGH,
                           preferred_element_type=jnp.float32)
        po[...] += jnp.dot(ohT, jnp.ones((RB, D), jnp.float32),
                           precision=_HIGH, preferred_element_type=jnp.float32)

        @pl.when(i == G1 - 1)
        def _():
            hc_ref[...] = (jnp.dot(ph[...], wr_ref[...], precision=_HIGH,
                                   preferred_element_type=jnp.float32)
                           + po[...] * b_ref[...])

    return pl.pallas_call(
        body,
        grid=(G1,),
        in_specs=[
            pl.BlockSpec((RB, D), lambda i: (i, 0)),
            pl.BlockSpec((1, 1, RB), lambda i: (i, 0, 0)),
            pl.BlockSpec((D, D), lambda i: (0, 0)),
            pl.BlockSpec((1, D), lambda i: (0, 0)),
        ],
        out_specs=pl.BlockSpec((B, D), lambda i: (0, 0)),
        out_shape=jax.ShapeDtypeStruct((B, D), jnp.float32),
        scratch_shapes=[
            pltpu.VMEM((B, D), jnp.float32),
            pltpu.VMEM((B, D), jnp.float32),
        ],
    )(h, seg3, WrT, b2)


def _tc_layer2(parts2, rinv, hc, seg3, WlT):
    def body(p_ref, rinv_ref, hc_ref, seg_ref, wl_ref, out_ref, pm):
        i = pl.program_id(0)

        @pl.when(i == 0)
        def _():
            pm[...] = jnp.zeros((B, D), jnp.float32)

        mean2 = (p_ref[0] + p_ref[1]) * rinv_ref[...]
        seg = seg_ref[0]
        ohT = (lax.broadcasted_iota(jnp.int32, (B, RB), 0) == seg
               ).astype(jnp.float32)
        pm[...] += jnp.dot(ohT, mean2, precision=_HIGH,
                           preferred_element_type=jnp.float32)

        @pl.when(i == G1 - 1)
        def _():
            out_ref[...] = jnp.dot(pm[...], wl_ref[...], precision=_HIGH,
                                   preferred_element_type=jnp.float32) + hc_ref[...]

    return pl.pallas_call(
        body,
        grid=(G1,),
        in_specs=[
            pl.BlockSpec((NC, RB, D), lambda i: (0, i, 0)),
            pl.BlockSpec((RB, D), lambda i: (i, 0)),
            pl.BlockSpec((B, D), lambda i: (0, 0)),
            pl.BlockSpec((1, 1, RB), lambda i: (i, 0, 0)),
            pl.BlockSpec((D, D), lambda i: (0, 0)),
        ],
        out_specs=pl.BlockSpec((B, D), lambda i: (0, 0)),
        out_shape=jax.ShapeDtypeStruct((B, D), jnp.float32),
        scratch_shapes=[pltpu.VMEM((B, D), jnp.float32)],
    )(parts2, rinv, hc, seg3, WlT)


def kernel(x, edge_index, batch, batch_size, Wl1, Wr1, b1, Wl2, Wr2, b2):
    x = x.astype(jnp.float32)
    ei = jnp.stack(
        [edge_index[0].astype(jnp.int32).reshape(NT, NCHUNK, CH),
         edge_index[1].astype(jnp.int32).reshape(NT, NCHUNK, CH)], axis=2)
    seg3 = jnp.minimum(batch, batch_size - 1).astype(jnp.int32).reshape(G1, 1, RB)

    xa = jnp.concatenate([x, jnp.ones((N, DC), jnp.float32)], axis=1)
    parts1 = _make_sc_agg(D + DC)(xa, ei)
    xr = _tc_xr(x, Wr1.T, b1.reshape(1, D))
    h, rinv = _tc_layer1(parts1, xr, Wl1.T)

    parts2 = _make_sc_agg(D)(h, ei)
    hc = _tc_pool_h(h, seg3, Wr2.T, b2.reshape(1, D))
    return _tc_layer2(parts2, rinv, hc, seg3, Wl2.T)


# speedup vs baseline: 1.8031x; 1.3680x over previous
import functools

import jax
import jax.numpy as jnp
from jax import lax
from jax.experimental import pallas as pl
from jax.experimental.pallas import tpu as pltpu
from jax.experimental.pallas import tpu_sc as plsc

N = 10000
E = 320000
D = 128
B = 64
DC = 16

NC = 2
NS = 16
NT = NC * NS
EPT = E // NT
CH = 40
NCHUNK = EPT // CH
RPT = N // NS
ZR = 25

RB = 1000
G1 = N // RB

_HIGH = lax.Precision.HIGHEST


@functools.lru_cache(maxsize=None)
def _make_sc_agg(with_count):
    mesh = plsc.VectorSubcoreMesh(core_axis_name="c", subcore_axis_name="s")
    width = D

    NR = 6
    NI = 12
    LG = 5
    LI = 8

    out_type = [jax.ShapeDtypeStruct((NC, N, width), jnp.float32)]
    if with_count:
        out_type.append(jax.ShapeDtypeStruct((NC, N, DC), jnp.float32))

    @functools.partial(
        pl.kernel,
        out_type=out_type,
        mesh=mesh,
        compiler_params=pltpu.CompilerParams(use_tc_tiling_on_sc=False),
        scratch_types=[
            pltpu.VMEM((NI, 2, CH), jnp.int32),
            pltpu.VMEM((NR, CH, width), jnp.float32),
            pltpu.VMEM((ZR, width), jnp.float32),
            pltpu.VMEM((CH, DC), jnp.float32),
            pltpu.VMEM_SHARED((N, width), jnp.float32),
            pltpu.VMEM_SHARED((N, DC), jnp.float32),
        ] + [pltpu.SemaphoreType.DMA] * (NI + 3 * NR),
    )
    def agg(x_hbm, ei_hbm, out_hbm, *rest):
        if with_count:
            cnt_hbm = rest[0]
            rest = rest[1:]
        idxr, rowsr, zbuf, ones, acc, cacc = rest[:6]
        sems = rest[6:]
        c = lax.axis_index("c")
        s = lax.axis_index("s")
        tid = c * NS + s
        ebase = tid * EPT
        isems = sems[:NI]
        gsems = sems[NI:NI + NR]
        ssems = sems[NI + NR:NI + 2 * NR]
        csems = sems[NI + 2 * NR:]

        def idx_start(ch, ia):
            off = ebase + ch * CH
            pltpu.async_copy(ei_hbm.at[0, pl.ds(off, CH)], idxr.at[ia, 0],
                             isems[ia])
            pltpu.async_copy(ei_hbm.at[1, pl.ds(off, CH)], idxr.at[ia, 1],
                             isems[ia])

        def idx_wait(ch, ia):
            off = ebase + ch * CH
            pltpu.make_async_copy(ei_hbm.at[0, pl.ds(off, CH)],
                                  idxr.at[ia, 0], isems[ia]).wait()
            pltpu.make_async_copy(ei_hbm.at[1, pl.ds(off, CH)],
                                  idxr.at[ia, 1], isems[ia]).wait()

        def g_start(rs, ia):
            pltpu.async_copy(x_hbm.at[idxr.at[ia, 0]], rowsr.at[rs],
                             gsems[rs])

        def g_wait(rs, ia):
            pltpu.make_async_copy(x_hbm.at[idxr.at[ia, 0]], rowsr.at[rs],
                                  gsems[rs]).wait()

        def s_start(rs, ia):
            pltpu.async_copy(rowsr.at[rs], acc.at[idxr.at[ia, 1]],
                             ssems[rs], add=True)
            if with_count:
                pltpu.async_copy(ones, cacc.at[idxr.at[ia, 1]],
                                 csems[rs], add=True)

        def s_wait(rs, ia):
            pltpu.make_async_copy(rowsr.at[rs], acc.at[idxr.at[ia, 1]],
                                  ssems[rs]).wait()
            if with_count:
                pltpu.make_async_copy(ones, cacc.at[idxr.at[ia, 1]],
                                      csems[rs]).wait()

        def step(k, j, guard):
            g_wait(j % NR, j % NI)
            s_start(j % NR, j % NI)
            if not guard or k + LG < NCHUNK:
                if not guard or k - (NR - LG) >= 0:
                    s_wait((j + LG) % NR, (j + LG - NR) % NI)
                idx_wait(k + LG, (j + LG) % NI)
                g_start((j + LG) % NR, (j + LG) % NI)
            if not guard or k + LI < NCHUNK:
                idx_start(k + LI, (j + LI) % NI)

        for k in range(LI):
            idx_start(k, k)
        for k in range(LG):
            idx_wait(k, k)
            g_start(k % NR, k)

        nv = width // 16

        def zrow(i, carry):
            for j in range(nv):
                zbuf[i, pl.ds(j * 16, 16)] = jnp.zeros((16,), jnp.float32)
            return carry

        lax.fori_loop(0, ZR, zrow, 0)

        def orow(i, carry):
            ones[i, pl.ds(0, 16)] = jnp.full((16,), 1.0, jnp.float32)
            return carry

        lax.fori_loop(0, CH, orow, 0)
        for r in range(RPT // ZR):
            pltpu.sync_copy(zbuf, acc.at[pl.ds(s * RPT + r * ZR, ZR)])
        if with_count:
            for r in range(RPT // ZR):
                pltpu.sync_copy(zbuf.at[:, pl.ds(0, DC)],
                                cacc.at[pl.ds(s * RPT + r * ZR, ZR)])
        plsc.subcore_barrier()

        for k in range(NI):
            step(k, k, True)

        def body(i, carry):
            k0 = NI * i
            for j in range(NI):
                step(k0 + j, j, False)
            return carry

        nbody = (NCHUNK - LI - NI) // NI
        lax.fori_loop(1, nbody + 1, body, 0)

        for k in range(NI * (nbody + 1), NCHUNK):
            step(k, k % NI, True)

        for k in range(NCHUNK - NR, NCHUNK):
            s_wait(k % NR, k % NI)
        plsc.subcore_barrier()

        pltpu.sync_copy(acc.at[pl.ds(s * RPT, RPT)],
                        out_hbm.at[c, pl.ds(s * RPT, RPT)])
        if with_count:
            pltpu.sync_copy(cacc.at[pl.ds(s * RPT, RPT)],
                            cnt_hbm.at[c, pl.ds(s * RPT, RPT)])

    return agg


def _tc_xr(x, WrT, b):
    def body(x_ref, wr_ref, b_ref, xr_ref):
        xr_ref[...] = jnp.dot(x_ref[...], wr_ref[...], precision=_HIGH,
                              preferred_element_type=jnp.float32) + b_ref[...]

    return pl.pallas_call(
        body,
        grid=(G1,),
        in_specs=[
            pl.BlockSpec((RB, D), lambda i: (i, 0)),
            pl.BlockSpec((D, D), lambda i: (0, 0)),
            pl.BlockSpec((1, D), lambda i: (0, 0)),
        ],
        out_specs=pl.BlockSpec((RB, D), lambda i: (i, 0)),
        out_shape=jax.ShapeDtypeStruct((N, D), jnp.float32),
    )(x, WrT, b)


def _tc_layer1(parts, cnt, xr, WlT):
    def body(p_ref, c_ref, xr_ref, wl_ref, h_ref, rinv_ref):
        agg = p_ref[0] + p_ref[1]
        cnt = c_ref[0, :, :1] + c_ref[1, :, :1]
        rinv = 1.0 / jnp.maximum(cnt, 1.0)
        mean = agg * rinv
        h = jnp.dot(mean, wl_ref[...], precision=_HIGH,
                    preferred_element_type=jnp.float32) + xr_ref[...]
        h_ref[...] = jnp.maximum(h, 0.0)
        rinv_ref[...] = jnp.broadcast_to(rinv, (RB, D))

    return pl.pallas_call(
        body,
        grid=(G1,),
        in_specs=[
            pl.BlockSpec((NC, RB, D), lambda i: (0, i, 0)),
            pl.BlockSpec((NC, RB, DC), lambda i: (0, i, 0)),
            pl.BlockSpec((RB, D), lambda i: (i, 0)),
            pl.BlockSpec((D, D), lambda i: (0, 0)),
        ],
        out_specs=[
            pl.BlockSpec((RB, D), lambda i: (i, 0)),
            pl.BlockSpec((RB, D), lambda i: (i, 0)),
        ],
        out_shape=[
            jax.ShapeDtypeStruct((N, D), jnp.float32),
            jax.ShapeDtypeStruct((N, D), jnp.float32),
        ],
    )(parts, cnt, xr, WlT)


def _tc_pool_h(h, seg3, WrT, b2):
    def body(h_ref, seg_ref, wr_ref, b_ref, hc_ref, ph, po):
        i = pl.program_id(0)

        @pl.when(i == 0)
        def _():
            ph[...] = jnp.zeros((B, D), jnp.float32)
            po[...] = jnp.zeros((B, D), jnp.float32)

        seg = jnp.minimum(seg_ref[0], B - 1)
        ohT = (lax.broadcasted_iota(jnp.int32, (B, RB), 0) == seg
               ).astype(jnp.float32)
        ph[...] += jnp.dot(ohT, h_ref[...], precision=_HIGH,
                           preferred_element_type=jnp.float32)
        po[...] += jnp.dot(ohT, jnp.ones((RB, D), jnp.float32),
                           precision=_HIGH, preferred_element_type=jnp.float32)

        @pl.when(i == G1 - 1)
        def _():
            hc_ref[...] = (jnp.dot(ph[...], wr_ref[...], precision=_HIGH,
                                   preferred_element_type=jnp.float32)
                           + po[...] * b_ref[...])

    return pl.pallas_call(
        body,
        grid=(G1,),
        in_specs=[
            pl.BlockSpec((RB, D), lambda i: (i, 0)),
            pl.BlockSpec((1, 1, RB), lambda i: (i, 0, 0)),
            pl.BlockSpec((D, D), lambda i: (0, 0)),
            pl.BlockSpec((1, D), lambda i: (0, 0)),
        ],
        out_specs=pl.BlockSpec((B, D), lambda i: (0, 0)),
        out_shape=jax.ShapeDtypeStruct((B, D), jnp.float32),
        scratch_shapes=[
            pltpu.VMEM((B, D), jnp.float32),
            pltpu.VMEM((B, D), jnp.float32),
        ],
    )(h, seg3, WrT, b2)


def _tc_layer2(parts2, rinv, hc, seg3, WlT):
    def body(p_ref, rinv_ref, hc_ref, seg_ref, wl_ref, out_ref, pm):
        i = pl.program_id(0)

        @pl.when(i == 0)
        def _():
            pm[...] = jnp.zeros((B, D), jnp.float32)

        mean2 = (p_ref[0] + p_ref[1]) * rinv_ref[...]
        seg = jnp.minimum(seg_ref[0], B - 1)
        ohT = (lax.broadcasted_iota(jnp.int32, (B, RB), 0) == seg
               ).astype(jnp.float32)
        pm[...] += jnp.dot(ohT, mean2, precision=_HIGH,
                           preferred_element_type=jnp.float32)

        @pl.when(i == G1 - 1)
        def _():
            out_ref[...] = jnp.dot(pm[...], wl_ref[...], precision=_HIGH,
                                   preferred_element_type=jnp.float32) + hc_ref[...]

    return pl.pallas_call(
        body,
        grid=(G1,),
        in_specs=[
            pl.BlockSpec((NC, RB, D), lambda i: (0, i, 0)),
            pl.BlockSpec((RB, D), lambda i: (i, 0)),
            pl.BlockSpec((B, D), lambda i: (0, 0)),
            pl.BlockSpec((1, 1, RB), lambda i: (i, 0, 0)),
            pl.BlockSpec((D, D), lambda i: (0, 0)),
        ],
        out_specs=pl.BlockSpec((B, D), lambda i: (0, 0)),
        out_shape=jax.ShapeDtypeStruct((B, D), jnp.float32),
        scratch_shapes=[pltpu.VMEM((B, D), jnp.float32)],
    )(parts2, rinv, hc, seg3, WlT)


def kernel(x, edge_index, batch, batch_size, Wl1, Wr1, b1, Wl2, Wr2, b2):
    x = x.astype(jnp.float32)
    ei = edge_index.astype(jnp.int32)
    seg3 = batch.astype(jnp.int32).reshape(G1, 1, RB)
    del batch_size

    parts1, cnt1 = _make_sc_agg(True)(x, ei)
    xr = _tc_xr(x, Wr1.T, b1.reshape(1, D))
    h, rinv = _tc_layer1(parts1, cnt1, xr, Wl1.T)

    parts2 = _make_sc_agg(False)(h, ei)[0]
    hc = _tc_pool_h(h, seg3, Wr2.T, b2.reshape(1, D))
    return _tc_layer2(parts2, rinv, hc, seg3, Wl2.T)
